# baseline (device time: 18446 ns/iter reference)
import jax
import jax.numpy as jnp
from jax import lax
from jax.experimental import pallas as pl
from jax.experimental.pallas import tpu as pltpu

N_DEV = 32


def kernel(A, B):
    m, _ = A.shape
    _, n = B.shape
    m_out = m // N_DEV

    def body(a_ref, b_ref, out_ref, chunks_ref, recv_ref, send_sems, recv_sems):
        my = lax.axis_index("i")

        barrier_sem = pltpu.get_barrier_semaphore()
        for k in range(1, N_DEV):
            p = lax.rem(my + k, N_DEV)
            pl.semaphore_signal(
                barrier_sem, inc=1,
                device_id=(p,), device_id_type=pl.DeviceIdType.MESH,
            )

        partial = jnp.dot(
            a_ref[...].astype(jnp.bfloat16),
            b_ref[...].astype(jnp.bfloat16),
            preferred_element_type=jnp.float32,
        )
        chunks_ref[...] = partial.reshape(N_DEV, m_out, n).astype(jnp.bfloat16)

        recv_ref[pl.ds(my, 1)] = chunks_ref[pl.ds(my, 1)]

        pl.semaphore_wait(barrier_sem, N_DEV - 1)

        sends = []
        for k in range(1, N_DEV):
            p = lax.rem(my + k, N_DEV)
            rdma = pltpu.make_async_remote_copy(
                src_ref=chunks_ref.at[p],
                dst_ref=recv_ref.at[my],
                send_sem=send_sems.at[k],
                recv_sem=recv_sems.at[my],
                device_id=(p,),
                device_id_type=pl.DeviceIdType.MESH,
            )
            rdma.start()
            sends.append(rdma)

        for k in range(1, N_DEV):
            q = lax.rem(my + k, N_DEV)
            recv = pltpu.make_async_remote_copy(
                src_ref=chunks_ref.at[q],
                dst_ref=recv_ref.at[q],
                send_sem=send_sems.at[0],
                recv_sem=recv_sems.at[q],
                device_id=(q,),
                device_id_type=pl.DeviceIdType.MESH,
            )
            recv.wait_recv()

        out_ref[...] = jnp.sum(recv_ref[...].astype(jnp.float32), axis=0)

        for rdma in sends:
            rdma.wait_send()

    return pl.pallas_call(
        body,
        out_shape=jax.ShapeDtypeStruct((m_out, n), jnp.float32),
        in_specs=[
            pl.BlockSpec(memory_space=pltpu.VMEM),
            pl.BlockSpec(memory_space=pltpu.VMEM),
        ],
        out_specs=pl.BlockSpec(memory_space=pltpu.VMEM),
        scratch_shapes=[
            pltpu.VMEM((N_DEV, m_out, n), jnp.bfloat16),
            pltpu.VMEM((N_DEV, m_out, n), jnp.bfloat16),
            pltpu.SemaphoreType.DMA((N_DEV,)),
            pltpu.SemaphoreType.DMA((N_DEV,)),
        ],
        compiler_params=pltpu.CompilerParams(collective_id=0),
    )(A, B)


# device time: 17823 ns/iter; 1.0350x vs baseline; 1.0350x over previous
import jax
import jax.numpy as jnp
from jax import lax
from jax.experimental import pallas as pl
from jax.experimental.pallas import tpu as pltpu

N_DEV = 32
G_SZ = 8
N_GRP = N_DEV // G_SZ


def kernel(A, B):
    m, _ = A.shape
    _, n = B.shape
    m_out = m // N_DEV

    def body(
        a_ref, b_ref, out_ref,
        chunks_ref, s1_recv_ref, gchunks_ref, s2_recv_ref,
        s1_send_sems, s1_recv_sems, s2_send_sems, s2_recv_sems,
    ):
        my = lax.axis_index("i")
        slot = lax.rem(my, G_SZ)
        grp = lax.div(my, G_SZ)
        base = my - slot

        barrier_sem = pltpu.get_barrier_semaphore()
        for k in range(1, G_SZ):
            pl.semaphore_signal(
                barrier_sem, inc=1,
                device_id=(base + lax.rem(slot + k, G_SZ),),
                device_id_type=pl.DeviceIdType.MESH,
            )
        for k in range(1, N_GRP):
            pl.semaphore_signal(
                barrier_sem, inc=1,
                device_id=(lax.rem(grp + k, N_GRP) * G_SZ + slot,),
                device_id_type=pl.DeviceIdType.MESH,
            )

        partial = jnp.dot(
            a_ref[...].astype(jnp.bfloat16),
            b_ref[...].astype(jnp.bfloat16),
            preferred_element_type=jnp.float32,
        )
        p4 = partial.reshape(N_GRP, G_SZ, m_out, n)
        chunks_ref[...] = jnp.transpose(p4, (1, 0, 2, 3)).astype(jnp.bfloat16)

        s1_recv_ref[pl.ds(slot, 1)] = chunks_ref[pl.ds(slot, 1)]

        pl.semaphore_wait(barrier_sem, G_SZ - 1 + N_GRP - 1)

        sends = []
        for k in range(1, G_SZ):
            j = lax.rem(slot + k, G_SZ)
            rdma = pltpu.make_async_remote_copy(
                src_ref=chunks_ref.at[j],
                dst_ref=s1_recv_ref.at[slot],
                send_sem=s1_send_sems.at[k],
                recv_sem=s1_recv_sems.at[slot],
                device_id=(base + j,),
                device_id_type=pl.DeviceIdType.MESH,
            )
            rdma.start()
            sends.append(rdma)

        for k in range(1, G_SZ):
            j = lax.rem(slot + k, G_SZ)
            recv = pltpu.make_async_remote_copy(
                src_ref=chunks_ref.at[j],
                dst_ref=s1_recv_ref.at[j],
                send_sem=s1_send_sems.at[0],
                recv_sem=s1_recv_sems.at[j],
                device_id=(base + j,),
                device_id_type=pl.DeviceIdType.MESH,
            )
            recv.wait_recv()

        gsum = jnp.sum(s1_recv_ref[...].astype(jnp.float32), axis=0)
        gchunks_ref[...] = gsum.astype(jnp.bfloat16)
        s2_recv_ref[pl.ds(grp, 1)] = gchunks_ref[pl.ds(grp, 1)]

        for k in range(1, N_GRP):
            g = lax.rem(grp + k, N_GRP)
            rdma = pltpu.make_async_remote_copy(
                src_ref=gchunks_ref.at[g],
                dst_ref=s2_recv_ref.at[grp],
                send_sem=s2_send_sems.at[k],
                recv_sem=s2_recv_sems.at[grp],
                device_id=(g * G_SZ + slot,),
                device_id_type=pl.DeviceIdType.MESH,
            )
            rdma.start()
            sends.append(rdma)

        for k in range(1, N_GRP):
            g = lax.rem(grp + k, N_GRP)
            recv = pltpu.make_async_remote_copy(
                src_ref=gchunks_ref.at[g],
                dst_ref=s2_recv_ref.at[g],
                send_sem=s2_send_sems.at[0],
                recv_sem=s2_recv_sems.at[g],
                device_id=(g * G_SZ + slot,),
                device_id_type=pl.DeviceIdType.MESH,
            )
            recv.wait_recv()

        out_ref[...] = jnp.sum(s2_recv_ref[...].astype(jnp.float32), axis=0)

        for rdma in sends:
            rdma.wait_send()

    return pl.pallas_call(
        body,
        out_shape=jax.ShapeDtypeStruct((m_out, n), jnp.float32),
        in_specs=[
            pl.BlockSpec(memory_space=pltpu.VMEM),
            pl.BlockSpec(memory_space=pltpu.VMEM),
        ],
        out_specs=pl.BlockSpec(memory_space=pltpu.VMEM),
        scratch_shapes=[
            pltpu.VMEM((G_SZ, N_GRP, m_out, n), jnp.bfloat16),
            pltpu.VMEM((G_SZ, N_GRP, m_out, n), jnp.bfloat16),
            pltpu.VMEM((N_GRP, m_out, n), jnp.bfloat16),
            pltpu.VMEM((N_GRP, m_out, n), jnp.bfloat16),
            pltpu.SemaphoreType.DMA((G_SZ,)),
            pltpu.SemaphoreType.DMA((G_SZ,)),
            pltpu.SemaphoreType.DMA((N_GRP,)),
            pltpu.SemaphoreType.DMA((N_GRP,)),
        ],
        compiler_params=pltpu.CompilerParams(collective_id=0),
    )(A, B)
